# baseline (speedup 1.0000x reference)
import functools

import jax
import jax.numpy as jnp
from jax import lax
from jax.experimental import pallas as pl
from jax.experimental.pallas import tpu as pltpu
from jax.experimental.pallas import tpu_sc as plsc

_EPS = 0.1
_N, _C, _W = 8192, 32000, 5
_NC, _NS, _L = 2, 16, 16
_NW = _NC * _NS
_RPW = _N // _NW
_TGT = _RPW
_WIN = _RPW * _W
_TOT = _TGT + _WIN
_CH = 128
_NCH = _TOT // _CH
_VPC = _CH // _L
_TPC = 1024
_ROWT = (_C // 128) * _TPC


def _loss_body(pred_hbm, tgt_hbm, smt_hbm, out_hbm,
               tgt_v, sm_v, idx_v, val_v, part_v, sem_i, sem_g):
    cid = lax.axis_index("c")
    sid = lax.axis_index("s")
    wid = sid * _NC + cid
    base = wid * _RPW

    cp_t = pltpu.async_copy(tgt_hbm.at[pl.ds(base, _RPW)], tgt_v, sem_i)
    cp_s = [
        pltpu.async_copy(smt_hbm.at[pl.ds(w * _N + base, _RPW)],
                         sm_v.at[pl.ds(w * _RPW, _RPW)], sem_i)
        for w in range(_W)
    ]

    lanes = lax.iota(jnp.int32, _L)
    row_part = (base >> 3) * _ROWT
    lane_rvec = (lanes >> 3) * _ROWT + (lanes & 7) * 128

    def chunk_body(col_ref, idx_off, ch_off, c):
        for k in range(_VPC):
            g = c * _VPC + k
            j16 = g * _L
            rvec16 = (g & (_RPW // _L - 1)) * (2 * _ROWT)
            cols = col_ref[pl.ds(j16, _L)]
            ci = cols & 127
            idx_v[pl.ds(idx_off + j16, _L)] = (
                row_part + rvec16 + lane_rvec + ((cols - ci) << 3) + ci)
        cc = ch_off + c
        pltpu.async_copy(
            pred_hbm.at[idx_v.at[pl.ds(cc * _CH, _CH)]],
            val_v.at[pl.ds(cc * _CH, _CH)], sem_g.at[cc])

    cp_t.wait()

    @plsc.parallel_loop(0, _TGT // _CH, unroll=2)
    def _(c):
        chunk_body(tgt_v, 0, 0, c)

    for cp in cp_s:
        cp.wait()

    @plsc.parallel_loop(0, _WIN // _CH, unroll=2)
    def _(c):
        chunk_body(sm_v, _TGT, _TGT // _CH, c)

    def dbody(c, accs):
        acc_t, acc_w = accs
        pltpu.make_async_copy(
            pred_hbm.at[idx_v.at[pl.ds(c * _CH, _CH)]],
            val_v.at[pl.ds(c * _CH, _CH)], sem_g.at[c]).wait()
        ch = jnp.zeros((_L,), jnp.float32)
        for k in range(_VPC):
            ch = ch + val_v[pl.ds(c * _CH + k * _L, _L)]
        zero = jnp.zeros((_L,), jnp.float32)
        is_t = c < _TGT // _CH
        return (acc_t + jnp.where(is_t, ch, zero),
                acc_w + jnp.where(is_t, zero, ch))

    acc_t, acc_w = lax.fori_loop(
        0, _NCH, dbody,
        (jnp.zeros((_L,), jnp.float32), jnp.zeros((_L,), jnp.float32)))
    part_v[...] = -(acc_t * ((1.0 - _EPS) / _N) + acc_w * (_EPS / (_N * _W)))
    pltpu.sync_copy(part_v, out_hbm.at[wid])


@functools.partial(
    pl.kernel,
    out_type=jax.ShapeDtypeStruct((_NW, _L), jnp.float32),
    mesh=plsc.VectorSubcoreMesh(core_axis_name="c", subcore_axis_name="s"),
    scratch_types=[
        pltpu.VMEM((_TGT,), jnp.int32),
        pltpu.VMEM((_WIN,), jnp.int32),
        pltpu.VMEM((_TOT,), jnp.int32),
        pltpu.VMEM((_TOT,), jnp.float32),
        pltpu.VMEM((_L,), jnp.float32),
        pltpu.SemaphoreType.DMA,
        pltpu.SemaphoreType.DMA((_NCH,)),
    ],
)
def _sc_loss(pred_hbm, tgt_hbm, smt_hbm, out_hbm,
             tgt_v, sm_v, idx_v, val_v, part_v, sem_i, sem_g):
    _loss_body(pred_hbm, tgt_hbm, smt_hbm, out_hbm,
               tgt_v, sm_v, idx_v, val_v, part_v, sem_i, sem_g)


def kernel(pred, target, smooth_idx):
    pred_flat = (pred.reshape(_N // 8, 8, _C // 128, 128)
                 .transpose(0, 2, 1, 3)
                 .reshape(-1))
    tgt = target.astype(jnp.int32)
    smt = smooth_idx.astype(jnp.int32).T.reshape(-1)
    parts = _sc_loss(pred_flat, tgt, smt)
    return jnp.sum(parts)


# speedup vs baseline: 1.0058x; 1.0058x over previous
import functools

import jax
import jax.numpy as jnp
from jax import lax
from jax.experimental import pallas as pl
from jax.experimental.pallas import tpu as pltpu
from jax.experimental.pallas import tpu_sc as plsc

_EPS = 0.1
_N, _C, _W = 8192, 32000, 5
_NC, _NS, _L = 2, 16, 16
_NW = _NC * _NS
_RPW = _N // _NW
_TGT = _RPW
_WIN = _RPW * _W
_TOT = _TGT + _WIN
_CH = 128
_NCH = _TOT // _CH
_VPC = _CH // _L
_TPC = 1024
_ROWT = (_C // 128) * _TPC


def _loss_body(pred_hbm, tgt_hbm, smt_hbm, out_hbm,
               tgt_v, sm_v, idx_v, val_v, part_v, sem_i, sem_g):
    cid = lax.axis_index("c")
    sid = lax.axis_index("s")
    wid = sid * _NC + cid
    base = wid * _RPW

    cp_t = pltpu.async_copy(tgt_hbm.at[pl.ds(base, _RPW)], tgt_v, sem_i)
    cp_s = [
        pltpu.async_copy(smt_hbm.at[pl.ds(w * _N + base, _RPW)],
                         sm_v.at[pl.ds(w * _RPW, _RPW)], sem_i)
        for w in range(_W)
    ]

    lanes = lax.iota(jnp.int32, _L)
    row_part = (base >> 3) * _ROWT
    lane_rvec = (lanes >> 3) * _ROWT + (lanes & 7) * 128

    def make_body(col_ref, idx_off, ch_off):
        def body(k, carry):
            j16 = k * _L
            rvec16 = (k & (_RPW // _L - 1)) * (2 * _ROWT)
            cols = col_ref[pl.ds(j16, _L)]
            ci = cols & 127
            idx_v[pl.ds(idx_off + j16, _L)] = (
                row_part + rvec16 + lane_rvec + ((cols - ci) << 3) + ci)

            @pl.when((k & (_VPC - 1)) == _VPC - 1)
            def _():
                c = ch_off + (k >> 3)
                pltpu.async_copy(
                    pred_hbm.at[idx_v.at[pl.ds(c * _CH, _CH)]],
                    val_v.at[pl.ds(c * _CH, _CH)], sem_g.at[c])

            return carry
        return body

    cp_t.wait()
    lax.fori_loop(0, _TGT // _L, make_body(tgt_v, 0, 0), 0)
    for cp in cp_s:
        cp.wait()
    lax.fori_loop(0, _WIN // _L, make_body(sm_v, _TGT, _TGT // _CH), 0)

    def dbody(c, accs):
        acc_t, acc_w = accs
        pltpu.make_async_copy(
            pred_hbm.at[idx_v.at[pl.ds(c * _CH, _CH)]],
            val_v.at[pl.ds(c * _CH, _CH)], sem_g.at[c]).wait()
        ch = jnp.zeros((_L,), jnp.float32)
        for k in range(_VPC):
            ch = ch + val_v[pl.ds(c * _CH + k * _L, _L)]
        zero = jnp.zeros((_L,), jnp.float32)
        is_t = c < _TGT // _CH
        return (acc_t + jnp.where(is_t, ch, zero),
                acc_w + jnp.where(is_t, zero, ch))

    acc_t, acc_w = lax.fori_loop(
        0, _NCH, dbody,
        (jnp.zeros((_L,), jnp.float32), jnp.zeros((_L,), jnp.float32)))
    part_v[...] = -(acc_t * ((1.0 - _EPS) / _N) + acc_w * (_EPS / (_N * _W)))
    pltpu.sync_copy(part_v, out_hbm.at[wid])


@functools.partial(
    pl.kernel,
    out_type=jax.ShapeDtypeStruct((_NW, _L), jnp.float32),
    mesh=plsc.VectorSubcoreMesh(core_axis_name="c", subcore_axis_name="s"),
    scratch_types=[
        pltpu.VMEM((_TGT,), jnp.int32),
        pltpu.VMEM((_WIN,), jnp.int32),
        pltpu.VMEM((_TOT,), jnp.int32),
        pltpu.VMEM((_TOT,), jnp.float32),
        pltpu.VMEM((_L,), jnp.float32),
        pltpu.SemaphoreType.DMA,
        pltpu.SemaphoreType.DMA((_NCH,)),
    ],
)
def _sc_loss(pred_hbm, tgt_hbm, smt_hbm, out_hbm,
             tgt_v, sm_v, idx_v, val_v, part_v, sem_i, sem_g):
    _loss_body(pred_hbm, tgt_hbm, smt_hbm, out_hbm,
               tgt_v, sm_v, idx_v, val_v, part_v, sem_i, sem_g)


def kernel(pred, target, smooth_idx):
    pred_flat = (pred.reshape(_N // 8, 8, _C // 128, 128)
                 .transpose(0, 2, 1, 3)
                 .reshape(-1))
    tgt = target.astype(jnp.int32)
    smt = smooth_idx.astype(jnp.int32).T.reshape(-1)
    parts = _sc_loss(pred_flat, tgt, smt)
    return jnp.sum(parts)
